# baseline (speedup 1.0000x reference)
import functools

import jax
import jax.numpy as jnp
from jax.experimental import pallas as pl
from jax.experimental.pallas import tpu as pltpu


def _ortho_norm(W):
    wtw = W.T @ W + 1e-4 * jnp.eye(W.shape[1], dtype=W.dtype)
    L = jnp.linalg.cholesky(wtw)
    return W @ jnp.linalg.inv(L).T


def _mm_post_body(flt_ref, a_ref, w_ref, o_ref):
    y = jnp.dot(flt_ref[...], a_ref[...], preferred_element_type=jnp.float32)
    o_ref[...] = jnp.dot(jnp.tanh(y), w_ref[...],
                         preferred_element_type=jnp.float32)


def _mm_tanh_body(flt_ref, a_ref, o_ref):
    y = jnp.dot(flt_ref[...], a_ref[...], preferred_element_type=jnp.float32)
    o_ref[...] = jnp.tanh(y)


def _stream_mm(flt, a, w=None, bm=400):
    n, k2 = flt.shape
    bm = min(bm, n)
    assert n % bm == 0
    ka = a.shape[1]
    grid = (n // bm,)
    in_specs = [
        pl.BlockSpec((bm, k2), lambda i: (i, 0)),
        pl.BlockSpec((k2, ka), lambda i: (0, 0)),
    ]
    operands = [flt, a]
    if w is None:
        body = _mm_tanh_body
        kb = ka
    else:
        body = _mm_post_body
        kb = w.shape[1]
        in_specs.append(pl.BlockSpec((ka, kb), lambda i: (0, 0)))
        operands.append(w)
    return pl.pallas_call(
        body,
        grid=grid,
        in_specs=in_specs,
        out_specs=pl.BlockSpec((bm, kb), lambda i: (i, 0)),
        out_shape=jax.ShapeDtypeStruct((n, kb), jnp.float32),
        compiler_params=pltpu.CompilerParams(
            dimension_semantics=("parallel",)),
    )(*operands)


def _proj_body(x_ref, w_ref, o_ref):
    o_ref[...] = jnp.dot(x_ref[...], w_ref[...],
                         preferred_element_type=jnp.float32)


def _proj(x, w):
    return pl.pallas_call(
        _proj_body,
        out_shape=jax.ShapeDtypeStruct((x.shape[0], w.shape[1]), jnp.float32),
    )(x, w)


def _fuse_body(h0_ref, h1_ref, u0_ref, u1_ref, h0c_ref, h1c_ref, hid_ref):
    h0c = h0_ref[...] - jnp.mean(h0_ref[...], axis=0, keepdims=True)
    h1c = h1_ref[...] - jnp.mean(h1_ref[...], axis=0, keepdims=True)
    h0c_ref[...] = h0c
    h1c_ref[...] = h1c
    hid_ref[...] = (jnp.dot(h0c, u0_ref[...], preferred_element_type=jnp.float32)
                    + jnp.dot(h1c, u1_ref[...], preferred_element_type=jnp.float32))


def _fuse(h0, h1, u0, u1):
    n, c = h0.shape
    return pl.pallas_call(
        _fuse_body,
        out_shape=(
            jax.ShapeDtypeStruct((n, c), jnp.float32),
            jax.ShapeDtypeStruct((n, c), jnp.float32),
            jax.ShapeDtypeStruct((n, u0.shape[1]), jnp.float32),
        ),
    )(h0, h1, u0, u1)


def kernel(feat0, feat1, flt0, flt1, flt_f, gc1_w0, gc2_w0, gc1_w1, gc2_w1,
           fus_w0, fus_w1):
    w10 = _ortho_norm(gc1_w0)
    w20 = _ortho_norm(gc2_w0)
    w11 = _ortho_norm(gc1_w1)
    w21 = _ortho_norm(gc2_w1)
    u0 = _ortho_norm(fus_w0)
    u1 = _ortho_norm(fus_w1)

    a0 = _proj(feat0, w10)
    a1 = _proj(feat1, w11)
    b0 = _stream_mm(flt0, a0, w=w20)
    b1 = _stream_mm(flt1, a1, w=w21)
    h0 = _stream_mm(flt0, b0)
    h1 = _stream_mm(flt1, b1)
    h0c, h1c, hidden = _fuse(h0, h1, u0, u1)
    common = _stream_mm(flt_f, hidden)
    return (common, h0c, h1c)
